# baseline (speedup 1.0000x reference)
import jax
import jax.numpy as jnp
from jax.experimental import pallas as pl
from jax.experimental.pallas import tpu as pltpu

_ACTION_S_IDX = 68
_ACTION_E_IDX = 88
_FULL_ACTION_DIM = 28
_BM = 2048


def _onehot_cols(idx_row, nfull, ncols):
    io_p = jax.lax.broadcasted_iota(jnp.int32, (nfull, ncols), 0)
    idx_b = jnp.broadcast_to(idx_row, (nfull, ncols))
    return (idx_b == io_p).astype(jnp.float32)


def _fused_body(obs_ref, prev_ref, w1_ref, b1_ref, w2_ref, b2_ref, w3_ref,
                b3_ref, waoi_ref, woi_ref, offs_ref, defs_ref, keep_ref,
                out_ref, w1s, w2s, w3ts, prevs, tbs, b1s, b2s):
    f32 = jnp.float32
    bf16 = jnp.bfloat16
    fp8 = jnp.float8_e4m3fn
    nact = w3_ref.shape[1]
    nfull = _FULL_ACTION_DIM
    outw = out_ref.shape[1]
    dimn = (((0,), (0,)), ((), ()))

    @pl.when(pl.program_id(0) == 0)
    def _init():
        keep24 = jnp.concatenate(
            [keep_ref[...],
             jnp.full((1, outw - keep_ref.shape[1]), -1, jnp.int32)], axis=1)
        OW = _onehot_cols(waoi_ref[...], nfull, nact)
        OD = _onehot_cols(woi_ref[...], nfull, nact)
        OK = _onehot_cols(keep24, nfull, outw)
        A = jax.lax.dot_general(OW, OK, dimn, preferred_element_type=f32)
        AD = jax.lax.dot_general(OD, OK, dimn, preferred_element_type=f32)
        hit = jnp.sum(A, axis=0, keepdims=True)
        cG = (jnp.dot(offs_ref[...], A, preferred_element_type=f32)
              + (1.0 - hit) * jnp.dot(defs_ref[...], AD,
                                      preferred_element_type=f32))
        A01 = A * 0.1
        w3ts[...] = jnp.dot(w3_ref[...], A01,
                            preferred_element_type=f32).astype(bf16)
        tbs[...] = jnp.dot(b3_ref[...], A01, preferred_element_type=f32) + cG

        io_r = jax.lax.broadcasted_iota(jnp.int32, (nact, obs_ref.shape[1]), 0)
        io_c = jax.lax.broadcasted_iota(jnp.int32, (nact, obs_ref.shape[1]), 1)
        SH = (io_c == io_r + _ACTION_S_IDX).astype(f32)
        prevs[...] = jnp.dot(prev_ref[...], SH, preferred_element_type=f32)

        w1s[...] = w1_ref[...].astype(fp8)
        w2s[...] = w2_ref[...].astype(fp8)
        b1s[...] = b1_ref[...].astype(bf16)
        b2s[...] = b2_ref[...].astype(bf16)

    obs = obs_ref[...]
    col = jax.lax.broadcasted_iota(jnp.int32, obs.shape, 1)
    in_seg = (col >= _ACTION_S_IDX) & (col < _ACTION_E_IDX)
    x = jnp.where(in_seg, prevs[...], obs).astype(fp8)
    h = jnp.tanh(jnp.dot(x, w1s[...],
                         preferred_element_type=f32).astype(bf16) + b1s[...])
    h = jnp.tanh(jnp.dot(h.astype(fp8), w2s[...],
                         preferred_element_type=f32).astype(bf16) + b2s[...])
    res = jnp.dot(h, w3ts[...], preferred_element_type=f32) + tbs[...]
    out_ref[...] = res


def kernel(obs, prev_full_action_wk, W1, b1, W2, b2, W3, b3,
           walking_action_out_indices, walking_offsets_indices,
           walking_offsets, walking_defaults, keep_mask):
    B, D = obs.shape
    H = W1.shape[1]
    nact = W3.shape[1]
    nkeep = keep_mask.shape[0]
    outw = nkeep + 4
    row = lambda v: v.reshape(1, -1)

    bm = min(_BM, B)
    full = lambda i: (0, 0)
    out = pl.pallas_call(
        _fused_body,
        grid=(pl.cdiv(B, bm),),
        in_specs=[
            pl.BlockSpec((bm, D), lambda i: (i, 0)),
            pl.BlockSpec((1, nact), full),
            pl.BlockSpec((D, H), full),
            pl.BlockSpec((1, H), full),
            pl.BlockSpec((H, H), full),
            pl.BlockSpec((1, H), full),
            pl.BlockSpec((H, nact), full),
            pl.BlockSpec((1, nact), full),
            pl.BlockSpec((1, nact), full),
            pl.BlockSpec((1, nact), full),
            pl.BlockSpec((1, nact), full),
            pl.BlockSpec((1, nact), full),
            pl.BlockSpec((1, nkeep), full),
        ],
        out_specs=pl.BlockSpec((bm, outw), lambda i: (i, 0)),
        out_shape=jax.ShapeDtypeStruct((B, outw), jnp.float32),
        scratch_shapes=[
            pltpu.VMEM((D, H), jnp.float8_e4m3fn),
            pltpu.VMEM((H, H), jnp.float8_e4m3fn),
            pltpu.VMEM((H, outw), jnp.bfloat16),
            pltpu.VMEM((1, D), jnp.float32),
            pltpu.VMEM((1, outw), jnp.float32),
            pltpu.VMEM((1, H), jnp.bfloat16),
            pltpu.VMEM((1, H), jnp.bfloat16),
        ],
        compiler_params=pltpu.CompilerParams(
            dimension_semantics=("arbitrary",)),
    )(obs, prev_full_action_wk, W1, row(b1), W2, row(b2), W3, row(b3),
      row(walking_action_out_indices), row(walking_offsets_indices),
      row(walking_offsets), row(walking_defaults), row(keep_mask))
    return out


# speedup vs baseline: 1.1127x; 1.1127x over previous
import jax
import jax.numpy as jnp
from jax.experimental import pallas as pl
from jax.experimental.pallas import tpu as pltpu

_ACTION_S_IDX = 68
_ACTION_E_IDX = 88
_FULL_ACTION_DIM = 28
_BM = 4096


def _onehot_cols(idx_row, nfull, ncols):
    io_p = jax.lax.broadcasted_iota(jnp.int32, (nfull, ncols), 0)
    idx_b = jnp.broadcast_to(idx_row, (nfull, ncols))
    return (idx_b == io_p).astype(jnp.float32)


def _fused_body(obs_ref, prev_ref, w1_ref, b1_ref, w2_ref, b2_ref, w3_ref,
                b3_ref, waoi_ref, woi_ref, offs_ref, defs_ref, keep_ref,
                out_ref, w1s, w2s, w3ts, prevs, tbs, b1s, b2s):
    f32 = jnp.float32
    bf16 = jnp.bfloat16
    fp8 = jnp.float8_e4m3fn
    nact = w3_ref.shape[1]
    nfull = _FULL_ACTION_DIM
    outw = out_ref.shape[1]
    dimn = (((0,), (0,)), ((), ()))

    @pl.when(pl.program_id(0) == 0)
    def _init():
        keep24 = jnp.concatenate(
            [keep_ref[...],
             jnp.full((1, outw - keep_ref.shape[1]), -1, jnp.int32)], axis=1)
        OW = _onehot_cols(waoi_ref[...], nfull, nact)
        OD = _onehot_cols(woi_ref[...], nfull, nact)
        OK = _onehot_cols(keep24, nfull, outw)
        A = jax.lax.dot_general(OW, OK, dimn, preferred_element_type=f32)
        AD = jax.lax.dot_general(OD, OK, dimn, preferred_element_type=f32)
        hit = jnp.sum(A, axis=0, keepdims=True)
        cG = (jnp.dot(offs_ref[...], A, preferred_element_type=f32)
              + (1.0 - hit) * jnp.dot(defs_ref[...], AD,
                                      preferred_element_type=f32))
        A01 = A * 0.1
        w3ts[...] = jnp.dot(w3_ref[...], A01,
                            preferred_element_type=f32).astype(bf16)
        tbs[...] = jnp.dot(b3_ref[...], A01, preferred_element_type=f32) + cG

        io_r = jax.lax.broadcasted_iota(jnp.int32, (nact, obs_ref.shape[1]), 0)
        io_c = jax.lax.broadcasted_iota(jnp.int32, (nact, obs_ref.shape[1]), 1)
        SH = (io_c == io_r + _ACTION_S_IDX).astype(f32)
        prevs[...] = jnp.dot(prev_ref[...], SH, preferred_element_type=f32)

        w1s[...] = w1_ref[...].astype(fp8)
        w2s[...] = w2_ref[...].astype(fp8)
        b1s[...] = b1_ref[...].astype(bf16)
        b2s[...] = b2_ref[...].astype(bf16)

    obs = obs_ref[...]
    col = jax.lax.broadcasted_iota(jnp.int32, obs.shape, 1)
    in_seg = (col >= _ACTION_S_IDX) & (col < _ACTION_E_IDX)
    x = jnp.where(in_seg, prevs[...], obs).astype(fp8)
    h = jnp.tanh(jnp.dot(x, w1s[...],
                         preferred_element_type=f32).astype(bf16) + b1s[...])
    h = jnp.tanh(jnp.dot(h.astype(fp8), w2s[...],
                         preferred_element_type=f32).astype(bf16) + b2s[...])
    res = jnp.dot(h.astype(fp8), w3ts[...].astype(fp8), preferred_element_type=f32) + tbs[...]
    out_ref[...] = res


def kernel(obs, prev_full_action_wk, W1, b1, W2, b2, W3, b3,
           walking_action_out_indices, walking_offsets_indices,
           walking_offsets, walking_defaults, keep_mask):
    B, D = obs.shape
    H = W1.shape[1]
    nact = W3.shape[1]
    nkeep = keep_mask.shape[0]
    outw = nkeep + 4
    row = lambda v: v.reshape(1, -1)

    bm = min(_BM, B)
    full = lambda i: (0, 0)
    out = pl.pallas_call(
        _fused_body,
        grid=(pl.cdiv(B, bm),),
        in_specs=[
            pl.BlockSpec((bm, D), lambda i: (i, 0)),
            pl.BlockSpec((1, nact), full),
            pl.BlockSpec((D, H), full),
            pl.BlockSpec((1, H), full),
            pl.BlockSpec((H, H), full),
            pl.BlockSpec((1, H), full),
            pl.BlockSpec((H, nact), full),
            pl.BlockSpec((1, nact), full),
            pl.BlockSpec((1, nact), full),
            pl.BlockSpec((1, nact), full),
            pl.BlockSpec((1, nact), full),
            pl.BlockSpec((1, nact), full),
            pl.BlockSpec((1, nkeep), full),
        ],
        out_specs=pl.BlockSpec((bm, outw), lambda i: (i, 0)),
        out_shape=jax.ShapeDtypeStruct((B, outw), jnp.float32),
        scratch_shapes=[
            pltpu.VMEM((D, H), jnp.float8_e4m3fn),
            pltpu.VMEM((H, H), jnp.float8_e4m3fn),
            pltpu.VMEM((H, outw), jnp.bfloat16),
            pltpu.VMEM((1, D), jnp.float32),
            pltpu.VMEM((1, outw), jnp.float32),
            pltpu.VMEM((1, H), jnp.bfloat16),
            pltpu.VMEM((1, H), jnp.bfloat16),
        ],
        compiler_params=pltpu.CompilerParams(
            dimension_semantics=("arbitrary",)),
    )(obs, prev_full_action_wk, W1, row(b1), W2, row(b2), W3, row(b3),
      row(walking_action_out_indices), row(walking_offsets_indices),
      row(walking_offsets), row(walking_defaults), row(keep_mask))
    return out
